# baseline (speedup 1.0000x reference)
import functools

import jax
import jax.numpy as jnp
from jax import lax
from jax.experimental import pallas as pl
from jax.experimental.pallas import tpu as pltpu
from jax.experimental.pallas import tpu_sc as plsc

CONF_T = 0.3
NMS_T = 0.5
MAXDET = 100
NEGV = -1e9
B = 8
N = 20000
LANES = 16
UNROLL = 25


def _decode_body(bb_ref, conf_ref, anc_ref, coords_ref, scores_ref,
                 areas_ref):
    dx = bb_ref[0]
    dy = bb_ref[1]
    dw = bb_ref[2]
    dh = bb_ref[3]
    acx = anc_ref[0]
    acy = anc_ref[1]
    aw = anc_ref[2]
    ah = anc_ref[3]
    cx = acx + dx * aw
    cy = acy + dy * ah
    w = aw * jnp.exp(jnp.clip(dw, -4.0, 4.0))
    h = ah * jnp.exp(jnp.clip(dh, -4.0, 4.0))
    x1 = cx - w / 2
    y1 = cy - h / 2
    x2 = cx + w / 2
    y2 = cy + h / 2
    coords_ref[0] = x1
    coords_ref[1] = y1
    coords_ref[2] = x2
    coords_ref[3] = y2
    areas_ref[...] = (x2 - x1) * (y2 - y1)
    s = jax.nn.sigmoid(conf_ref[...])
    scores_ref[...] = jnp.where(s > CONF_T, s, NEGV)


def _decode(bb_t, conf_pred, anc_t):
    return pl.pallas_call(
        _decode_body,
        out_shape=[
            jax.ShapeDtypeStruct((4, B, N), jnp.float32),
            jax.ShapeDtypeStruct((B, N), jnp.float32),
            jax.ShapeDtypeStruct((B, N), jnp.float32),
        ],
    )(bb_t, conf_pred, anc_t)


def _nms_body(coords_hbm, scores_hbm, areas_hbm, out_hbm,
              x1, y1, x2, y2, sc, ar, outb):
    wid = lax.axis_index("s") * 2 + lax.axis_index("c")

    @pl.when(wid < B)
    def _():
        img = wid
        pltpu.sync_copy(coords_hbm.at[0, img], x1)
        pltpu.sync_copy(coords_hbm.at[1, img], y1)
        pltpu.sync_copy(coords_hbm.at[2, img], x2)
        pltpu.sync_copy(coords_hbm.at[3, img], y2)
        pltpu.sync_copy(scores_hbm.at[img], sc)
        pltpu.sync_copy(areas_hbm.at[img], ar)

        iota = lax.iota(jnp.int32, 16)
        negv = jnp.full((16,), NEGV, jnp.float32)
        init = (jnp.full((16,), -2e9, jnp.float32),
                jnp.zeros((16,), jnp.int32))

        gdn = lax.GatherDimensionNumbers(
            offset_dims=(), collapsed_slice_dims=(0,), start_index_map=(0,))

        def lane_perm(v, perm):
            return lax.gather(v, perm[:, None], gdn, slice_sizes=(1,),
                              mode=lax.GatherScatterMode.PROMISE_IN_BOUNDS)

        def reduce_best(best, bidx):
            for sh in (8, 4, 2, 1):
                perm = jnp.bitwise_and(iota + sh, 15)
                bv = lane_perm(best, perm)
                bi = lane_perm(bidx, perm)
                take = (bv > best) | ((bv == best) & (bi < bidx))
                best = jnp.where(take, bv, best)
                bidx = jnp.where(take, bi, bidx)
            return best, bidx

        @plsc.parallel_loop(0, N, step=LANES, unroll=UNROLL, carry=init)
        def amax0(base, carry):
            best, bidx = carry
            v = sc[pl.ds(base, LANES)]
            idx = iota + base
            gt = v > best
            best = jnp.where(gt, v, best)
            bidx = jnp.where(gt, idx, bidx)
            return best, bidx

        gmax0, gidx0 = reduce_best(*amax0)

        def step(k, carry):
            gmaxv, gidxv = carry
            bx1 = plsc.load_gather(x1, [gidxv])
            by1 = plsc.load_gather(y1, [gidxv])
            bx2 = plsc.load_gather(x2, [gidxv])
            by2 = plsc.load_gather(y2, [gidxv])
            barea = plsc.load_gather(ar, [gidxv])

            validv = gmaxv > CONF_T
            row = jnp.where(iota == 0, bx1, 0.0)
            row = jnp.where(iota == 1, by1, row)
            row = jnp.where(iota == 2, bx2, row)
            row = jnp.where(iota == 3, by2, row)
            row = jnp.where(iota == 4, gmaxv, row)
            row = jnp.where(validv, row, 0.0)
            outb[pl.ds(k * 16, 16)] = row

            @plsc.parallel_loop(0, N, step=LANES, unroll=UNROLL,
                                carry=init)
            def sweep(base, carry2):
                best, bidx = carry2
                x1c = x1[pl.ds(base, LANES)]
                y1c = y1[pl.ds(base, LANES)]
                x2c = x2[pl.ds(base, LANES)]
                y2c = y2[pl.ds(base, LANES)]
                scc = sc[pl.ds(base, LANES)]
                areac = ar[pl.ds(base, LANES)]
                xx1 = jnp.maximum(bx1, x1c)
                yy1 = jnp.maximum(by1, y1c)
                xx2 = jnp.minimum(bx2, x2c)
                yy2 = jnp.minimum(by2, y2c)
                inter = (jnp.maximum(xx2 - xx1, 0.0)
                         * jnp.maximum(yy2 - yy1, 0.0))
                iou = inter / (areac + barea - inter + 1e-6)
                scn = jnp.where(iou > NMS_T, negv, scc)
                sc[pl.ds(base, LANES)] = scn
                idx = iota + base
                gt = scn > best
                best = jnp.where(gt, scn, best)
                bidx = jnp.where(gt, idx, bidx)
                return best, bidx

            return reduce_best(*sweep)

        lax.fori_loop(0, MAXDET, step, (gmax0, gidx0))
        pltpu.sync_copy(outb, out_hbm.at[img])


def _nms(coords, scores, areas):
    mesh = plsc.VectorSubcoreMesh(core_axis_name="c", subcore_axis_name="s")
    f = functools.partial(
        pl.kernel,
        mesh=mesh,
        compiler_params=pltpu.CompilerParams(needs_layout_passes=False),
        out_type=jax.ShapeDtypeStruct((B, MAXDET * 16), jnp.float32),
        scratch_types=[
            pltpu.VMEM((N,), jnp.float32),
            pltpu.VMEM((N,), jnp.float32),
            pltpu.VMEM((N,), jnp.float32),
            pltpu.VMEM((N,), jnp.float32),
            pltpu.VMEM((N,), jnp.float32),
            pltpu.VMEM((N,), jnp.float32),
            pltpu.VMEM((MAXDET * 16,), jnp.float32),
        ],
    )(_nms_body)
    return f(coords, scores, areas)


def kernel(bbox_pred, conf_pred, anchors):
    bb_t = jnp.transpose(bbox_pred, (2, 0, 1))
    anc_t = jnp.transpose(anchors, (1, 0))[:, None, :]
    coords, scores, areas = _decode(bb_t, conf_pred, anc_t)
    out = _nms(coords, scores, areas)
    return out.reshape(B, MAXDET, 16)[:, :, :5]


# speedup vs baseline: 3.6806x; 3.6806x over previous
import functools

import jax
import jax.numpy as jnp
from jax import lax
from jax.experimental import pallas as pl
from jax.experimental.pallas import tpu as pltpu
from jax.experimental.pallas import tpu_sc as plsc

CONF_T = 0.3
NMS_T = 0.5
MAXDET = 100
NEGV = -1e9
B = 8
N = 20000
LANES = 16
UNROLL = 5


def _decode_body(bb_ref, conf_ref, anc_ref, coords_ref, scores_ref,
                 areas_ref):
    dx = bb_ref[0]
    dy = bb_ref[1]
    dw = bb_ref[2]
    dh = bb_ref[3]
    acx = anc_ref[0]
    acy = anc_ref[1]
    aw = anc_ref[2]
    ah = anc_ref[3]
    cx = acx + dx * aw
    cy = acy + dy * ah
    w = aw * jnp.exp(jnp.clip(dw, -4.0, 4.0))
    h = ah * jnp.exp(jnp.clip(dh, -4.0, 4.0))
    x1 = cx - w / 2
    y1 = cy - h / 2
    x2 = cx + w / 2
    y2 = cy + h / 2
    coords_ref[0] = x1
    coords_ref[1] = y1
    coords_ref[2] = x2
    coords_ref[3] = y2
    areas_ref[...] = (x2 - x1) * (y2 - y1)
    s = jax.nn.sigmoid(conf_ref[...])
    scores_ref[...] = jnp.where(s > CONF_T, s, NEGV)


def _decode(bb_t, conf_pred, anc_t):
    return pl.pallas_call(
        _decode_body,
        out_shape=[
            jax.ShapeDtypeStruct((4, B, N), jnp.float32),
            jax.ShapeDtypeStruct((B, N), jnp.float32),
            jax.ShapeDtypeStruct((B, N), jnp.float32),
        ],
    )(bb_t, conf_pred, anc_t)


def _nms_body(coords_hbm, scores_hbm, areas_hbm, out_hbm,
              x1, y1, x2, y2, sc, ar, outb):
    wid = lax.axis_index("s") * 2 + lax.axis_index("c")

    @pl.when(wid < B)
    def _():
        img = wid
        pltpu.sync_copy(coords_hbm.at[0, img], x1)
        pltpu.sync_copy(coords_hbm.at[1, img], y1)
        pltpu.sync_copy(coords_hbm.at[2, img], x2)
        pltpu.sync_copy(coords_hbm.at[3, img], y2)
        pltpu.sync_copy(scores_hbm.at[img], sc)
        pltpu.sync_copy(areas_hbm.at[img], ar)

        iota = lax.iota(jnp.int32, 16)
        negv = jnp.full((16,), NEGV, jnp.float32)
        init = (jnp.full((16,), -2e9, jnp.float32),
                jnp.zeros((16,), jnp.int32))

        gdn = lax.GatherDimensionNumbers(
            offset_dims=(), collapsed_slice_dims=(0,), start_index_map=(0,))

        def lane_perm(v, perm):
            return lax.gather(v, perm[:, None], gdn, slice_sizes=(1,),
                              mode=lax.GatherScatterMode.PROMISE_IN_BOUNDS)

        def reduce_best(best, bidx):
            for sh in (8, 4, 2, 1):
                perm = jnp.bitwise_and(iota + sh, 15)
                bv = lane_perm(best, perm)
                bi = lane_perm(bidx, perm)
                take = (bv > best) | ((bv == best) & (bi < bidx))
                best = jnp.where(take, bv, best)
                bidx = jnp.where(take, bi, bidx)
            return best, bidx

        @plsc.parallel_loop(0, N, step=LANES, unroll=UNROLL, carry=init)
        def amax0(base, carry):
            best, bidx = carry
            v = sc[pl.ds(base, LANES)]
            idx = iota + base
            gt = v > best
            best = jnp.where(gt, v, best)
            bidx = jnp.where(gt, idx, bidx)
            return best, bidx

        gmax0, gidx0 = reduce_best(*amax0)

        def step(k, carry):
            gmaxv, gidxv = carry
            bx1 = plsc.load_gather(x1, [gidxv])
            by1 = plsc.load_gather(y1, [gidxv])
            bx2 = plsc.load_gather(x2, [gidxv])
            by2 = plsc.load_gather(y2, [gidxv])
            barea = plsc.load_gather(ar, [gidxv])

            validv = gmaxv > CONF_T
            row = jnp.where(iota == 0, bx1, 0.0)
            row = jnp.where(iota == 1, by1, row)
            row = jnp.where(iota == 2, bx2, row)
            row = jnp.where(iota == 3, by2, row)
            row = jnp.where(iota == 4, gmaxv, row)
            row = jnp.where(validv, row, 0.0)
            outb[pl.ds(k * 16, 16)] = row

            @plsc.parallel_loop(0, N, step=LANES, unroll=UNROLL,
                                carry=init)
            def sweep(base, carry2):
                best, bidx = carry2
                x1c = x1[pl.ds(base, LANES)]
                y1c = y1[pl.ds(base, LANES)]
                x2c = x2[pl.ds(base, LANES)]
                y2c = y2[pl.ds(base, LANES)]
                scc = sc[pl.ds(base, LANES)]
                areac = ar[pl.ds(base, LANES)]
                xx1 = jnp.maximum(bx1, x1c)
                yy1 = jnp.maximum(by1, y1c)
                xx2 = jnp.minimum(bx2, x2c)
                yy2 = jnp.minimum(by2, y2c)
                inter = (jnp.maximum(xx2 - xx1, 0.0)
                         * jnp.maximum(yy2 - yy1, 0.0))
                iou = inter / (areac + barea - inter + 1e-6)
                scn = jnp.where(iou > NMS_T, negv, scc)
                sc[pl.ds(base, LANES)] = scn
                idx = iota + base
                gt = scn > best
                best = jnp.where(gt, scn, best)
                bidx = jnp.where(gt, idx, bidx)
                return best, bidx

            return reduce_best(*sweep)

        lax.fori_loop(0, MAXDET, step, (gmax0, gidx0))
        pltpu.sync_copy(outb, out_hbm.at[img])


def _nms(coords, scores, areas):
    mesh = plsc.VectorSubcoreMesh(core_axis_name="c", subcore_axis_name="s")
    f = functools.partial(
        pl.kernel,
        mesh=mesh,
        compiler_params=pltpu.CompilerParams(needs_layout_passes=False),
        out_type=jax.ShapeDtypeStruct((B, MAXDET * 16), jnp.float32),
        scratch_types=[
            pltpu.VMEM((N,), jnp.float32),
            pltpu.VMEM((N,), jnp.float32),
            pltpu.VMEM((N,), jnp.float32),
            pltpu.VMEM((N,), jnp.float32),
            pltpu.VMEM((N,), jnp.float32),
            pltpu.VMEM((N,), jnp.float32),
            pltpu.VMEM((MAXDET * 16,), jnp.float32),
        ],
    )(_nms_body)
    return f(coords, scores, areas)


def kernel(bbox_pred, conf_pred, anchors):
    bb_t = jnp.transpose(bbox_pred, (2, 0, 1))
    anc_t = jnp.transpose(anchors, (1, 0))[:, None, :]
    coords, scores, areas = _decode(bb_t, conf_pred, anc_t)
    out = _nms(coords, scores, areas)
    return out.reshape(B, MAXDET, 16)[:, :, :5]
